# Initial kernel scaffold; baseline (speedup 1.0000x reference)
import jax
import jax.numpy as jnp
from jax.experimental import pallas as pl


def kernel(x, edge_index, W1, b1, Wmu, bmu):
    raise NotImplementedError("write your pallas kernel here")


# speedup vs baseline: 2.8626x; 2.8626x over previous
import functools

import jax
import jax.numpy as jnp
from jax import lax
from jax.experimental import pallas as pl
from jax.experimental.pallas import tpu as pltpu

_N = 10000
_E = 320000
_ROWS = 1000


def _mm_scale_body(x_ref, w_ref, dinv_ref, hs_ref):
    h = jnp.dot(x_ref[...], w_ref[...], preferred_element_type=jnp.float32)
    hs_ref[...] = h * dinv_ref[...]


def _mm_scale(x, w, dinv):
    n, _ = x.shape
    dout = w.shape[1]
    return pl.pallas_call(
        _mm_scale_body,
        grid=(n // _ROWS,),
        in_specs=[
            pl.BlockSpec((_ROWS, x.shape[1]), lambda i: (i, 0)),
            pl.BlockSpec((w.shape[0], dout), lambda i: (0, 0)),
            pl.BlockSpec((_ROWS, 1), lambda i: (i, 0)),
        ],
        out_specs=pl.BlockSpec((_ROWS, dout), lambda i: (i, 0)),
        out_shape=jax.ShapeDtypeStruct((n, dout), jnp.float32),
    )(x, w, dinv)


def _ep_mm_body(agg_ref, hs_ref, b_ref, dinv_ref, w_ref, out_ref):
    dinv = dinv_ref[...]
    h = jnp.maximum(dinv * (agg_ref[...] + hs_ref[...]) + b_ref[...], 0.0)
    out_ref[...] = jnp.dot(h, w_ref[...], preferred_element_type=jnp.float32) * dinv


def _ep_mm(agg, hs, b, dinv, w):
    n, din = agg.shape
    dout = w.shape[1]
    return pl.pallas_call(
        _ep_mm_body,
        grid=(n // _ROWS,),
        in_specs=[
            pl.BlockSpec((_ROWS, din), lambda i: (i, 0)),
            pl.BlockSpec((_ROWS, din), lambda i: (i, 0)),
            pl.BlockSpec((1, din), lambda i: (0, 0)),
            pl.BlockSpec((_ROWS, 1), lambda i: (i, 0)),
            pl.BlockSpec((din, dout), lambda i: (0, 0)),
        ],
        out_specs=pl.BlockSpec((_ROWS, dout), lambda i: (i, 0)),
        out_shape=jax.ShapeDtypeStruct((n, dout), jnp.float32),
    )(agg, hs, b, dinv, w)


def _final_body(agg_ref, hs_ref, b_ref, dinv_ref, eps_ref, mu_ref, zeta_ref):
    mu = dinv_ref[...] * (agg_ref[...] + hs_ref[...]) + b_ref[...]
    mu_ref[...] = mu
    zeta_ref[...] = mu + eps_ref[...] * jnp.exp(mu)


def _final(agg, hs, b, dinv, eps):
    n, d = agg.shape
    return pl.pallas_call(
        _final_body,
        grid=(n // _ROWS,),
        in_specs=[
            pl.BlockSpec((_ROWS, d), lambda i: (i, 0)),
            pl.BlockSpec((_ROWS, d), lambda i: (i, 0)),
            pl.BlockSpec((1, d), lambda i: (0, 0)),
            pl.BlockSpec((_ROWS, 1), lambda i: (i, 0)),
            pl.BlockSpec((_ROWS, d), lambda i: (i, 0)),
        ],
        out_specs=[
            pl.BlockSpec((_ROWS, d), lambda i: (i, 0)),
            pl.BlockSpec((_ROWS, d), lambda i: (i, 0)),
        ],
        out_shape=[
            jax.ShapeDtypeStruct((n, d), jnp.float32),
            jax.ShapeDtypeStruct((n, d), jnp.float32),
        ],
    )(agg, hs, b, dinv, eps)


def kernel(x, edge_index, W1, b1, Wmu, bmu):
    src = edge_index[0]
    dst = edge_index[1]

    indeg = jnp.zeros((_N,), jnp.float32).at[dst].add(1.0)
    dinv = lax.rsqrt(indeg + 1.0)[:, None]

    hs1 = _mm_scale(x, W1, dinv)
    agg1 = jnp.zeros_like(hs1).at[dst].add(hs1[src])
    hs2 = _ep_mm(agg1, hs1, b1[None, :], dinv, Wmu)
    agg2 = jnp.zeros_like(hs2).at[dst].add(hs2[src])
    eps = jax.random.normal(jax.random.key(42), hs2.shape, dtype=jnp.float32)
    mu, zeta = _final(agg2, hs2, bmu[None, :], dinv, eps)
    return (mu, mu, zeta)


# speedup vs baseline: 9.8360x; 3.4361x over previous
import functools

import jax
import jax.numpy as jnp
from jax import lax
from jax.experimental import pallas as pl
from jax.experimental.pallas import tpu as pltpu
from jax.experimental.pallas import tpu_sc as plsc

_N = 10000
_E = 320000
_NC = 2
_NS = 16
_K = 80
_ROWS = 1000

_sc_mesh = plsc.VectorSubcoreMesh(core_axis_name="c", subcore_axis_name="s")


def _deg_body(dst_hbm, out_hbm, dstv, ones, zbuf, hist):
    s = lax.axis_index("s")
    c = lax.axis_index("c")
    for i in range(_K // 16):
        ones[pl.ds(i * 16, 16)] = jnp.ones((16,), jnp.float32)
    for i in range(640 // 16):
        zbuf[pl.ds(i * 16, 16)] = jnp.zeros((16,), jnp.float32)
    @pl.when(s < 15)
    def _():
        pltpu.sync_copy(zbuf.at[pl.ds(0, 624)], hist.at[pl.ds(s * 624, 624)])

    @pl.when(s == 15)
    def _():
        pltpu.sync_copy(zbuf, hist.at[pl.ds(9360, 640)])

    plsc.subcore_barrier()
    ep = _E // _NS

    def body(j, carry):
        off = s * ep + j * _K
        pltpu.sync_copy(dst_hbm.at[pl.ds(off, _K)], dstv)
        pltpu.sync_copy(ones, hist.at[dstv], add=True)
        return carry

    lax.fori_loop(0, ep // _K, body, 0)
    plsc.subcore_barrier()
    t = c * _NS + s
    @pl.when(t < 31)
    def _():
        pltpu.sync_copy(hist.at[pl.ds(t * 320, 320)], zbuf.at[pl.ds(0, 320)])
        pltpu.sync_copy(zbuf.at[pl.ds(0, 320)], out_hbm.at[pl.ds(t * 320, 320)])

    @pl.when(t == 31)
    def _():
        pltpu.sync_copy(hist.at[pl.ds(31 * 320, 80)], zbuf.at[pl.ds(0, 80)])
        pltpu.sync_copy(zbuf.at[pl.ds(0, 80)], out_hbm.at[pl.ds(31 * 320, 80)])


_deg = functools.partial(
    pl.kernel,
    _deg_body,
    out_type=jax.ShapeDtypeStruct((_N,), jnp.float32),
    mesh=_sc_mesh,
    scratch_types=[
        pltpu.VMEM((_K,), jnp.int32),
        pltpu.VMEM((_K,), jnp.float32),
        pltpu.VMEM((640,), jnp.float32),
        pltpu.VMEM_SHARED((_N,), jnp.float32),
    ],
)()


def _zero_acc(s, zbuf, acc):
    for r in range(8):
        for i in range(128 // 16):
            zbuf[r, pl.ds(i * 16, 16)] = jnp.zeros((16,), jnp.float32)
    nz = jnp.where(s < 15, 78, 80)

    def zbody(j, carry):
        pltpu.sync_copy(zbuf, acc.at[pl.ds(s * 624 + j * 8, 8)])
        return carry

    lax.fori_loop(0, nz, zbody, 0)


def _edge_loop(base, niter, table, src_hbm, dst_hbm, srcv, dstv, rows, acc,
               sem):
    def body(j, carry):
        off = base + j * _K
        pltpu.sync_copy(src_hbm.at[pl.ds(off, _K)], srcv)
        pltpu.sync_copy(dst_hbm.at[pl.ds(off, _K)], dstv)
        pltpu.async_copy(table.at[srcv], rows, sem).wait()
        pltpu.sync_copy(rows, acc.at[dstv], add=True)
        return carry

    lax.fori_loop(0, niter, body, 0)


def _drain_acc(s, stage, acc, out2d):
    def obody(j, carry):
        r = s * 624 + j * 104
        pltpu.sync_copy(acc.at[pl.ds(r, 104)], stage.at[pl.ds(0, 104)])
        pltpu.sync_copy(stage.at[pl.ds(0, 104)], out2d.at[pl.ds(r, 104)])
        return carry

    lax.fori_loop(0, 6, obody, 0)
    @pl.when(s == 15)
    def _():
        pltpu.sync_copy(acc.at[pl.ds(9984, 16)], stage.at[pl.ds(0, 16)])
        pltpu.sync_copy(stage.at[pl.ds(0, 16)], out2d.at[pl.ds(9984, 16)])


def _agg1_body(table_hbm, src_hbm, dst_hbm, out_hbm,
               srcv, dstv, rows, zbuf, stage, acc, sem):
    s = lax.axis_index("s")
    c = lax.axis_index("c")
    _zero_acc(s, zbuf, acc)
    plsc.subcore_barrier()
    col = pl.multiple_of(c * 128, 128)
    tcol = table_hbm.at[:, pl.ds(col, 128)]
    ep = _E // _NS
    _edge_loop(s * ep, ep // _K, tcol, src_hbm, dst_hbm,
               srcv, dstv, rows, acc, sem)
    plsc.subcore_barrier()
    _drain_acc(s, stage, acc, out_hbm.at[:, pl.ds(col, 128)])


_agg1 = functools.partial(
    pl.kernel,
    _agg1_body,
    out_type=jax.ShapeDtypeStruct((_N, 256), jnp.float32),
    mesh=_sc_mesh,
    scratch_types=[
        pltpu.VMEM((_K,), jnp.int32),
        pltpu.VMEM((_K,), jnp.int32),
        pltpu.VMEM((_K, 128), jnp.float32),
        pltpu.VMEM((8, 128), jnp.float32),
        pltpu.VMEM((104, 128), jnp.float32),
        pltpu.VMEM_SHARED((_N, 128), jnp.float32),
        pltpu.SemaphoreType.DMA,
    ],
)()


def _agg2_body(table_hbm, src_hbm, dst_hbm, out_hbm,
               srcv, dstv, rows, zbuf, stage, acc, sem):
    s = lax.axis_index("s")
    c = lax.axis_index("c")
    _zero_acc(s, zbuf, acc)
    plsc.subcore_barrier()
    ep = _E // (_NC * _NS)
    _edge_loop((c * _NS + s) * ep, ep // _K, table_hbm, src_hbm, dst_hbm,
               srcv, dstv, rows, acc, sem)
    plsc.subcore_barrier()
    _drain_acc(s, stage, acc, out_hbm.at[c])


_agg2 = functools.partial(
    pl.kernel,
    _agg2_body,
    out_type=jax.ShapeDtypeStruct((_NC, _N, 128), jnp.float32),
    mesh=_sc_mesh,
    scratch_types=[
        pltpu.VMEM((_K,), jnp.int32),
        pltpu.VMEM((_K,), jnp.int32),
        pltpu.VMEM((_K, 128), jnp.float32),
        pltpu.VMEM((8, 128), jnp.float32),
        pltpu.VMEM((104, 128), jnp.float32),
        pltpu.VMEM_SHARED((_N, 128), jnp.float32),
        pltpu.SemaphoreType.DMA,
    ],
)()


def _mm_scale_body(x_ref, w_ref, indeg_ref, hs_ref, dinv_ref):
    dinv = lax.rsqrt(indeg_ref[...] + 1.0)
    h = jnp.dot(x_ref[...], w_ref[...], preferred_element_type=jnp.float32)
    hs_ref[...] = h * dinv
    dinv_ref[...] = dinv


def _mm_scale(x, w, indeg):
    n, _ = x.shape
    dout = w.shape[1]
    return pl.pallas_call(
        _mm_scale_body,
        grid=(n // _ROWS,),
        in_specs=[
            pl.BlockSpec((_ROWS, x.shape[1]), lambda i: (i, 0)),
            pl.BlockSpec((w.shape[0], dout), lambda i: (0, 0)),
            pl.BlockSpec((_ROWS, 1), lambda i: (i, 0)),
        ],
        out_specs=[
            pl.BlockSpec((_ROWS, dout), lambda i: (i, 0)),
            pl.BlockSpec((_ROWS, 1), lambda i: (i, 0)),
        ],
        out_shape=[
            jax.ShapeDtypeStruct((n, dout), jnp.float32),
            jax.ShapeDtypeStruct((n, 1), jnp.float32),
        ],
    )(x, w, indeg)


def _ep_mm_body(agg_ref, hs_ref, b_ref, dinv_ref, w_ref, out_ref):
    dinv = dinv_ref[...]
    h = jnp.maximum(dinv * (agg_ref[...] + hs_ref[...]) + b_ref[...], 0.0)
    out_ref[...] = jnp.dot(h, w_ref[...], preferred_element_type=jnp.float32) * dinv


def _ep_mm(agg, hs, b, dinv, w):
    n, din = agg.shape
    dout = w.shape[1]
    return pl.pallas_call(
        _ep_mm_body,
        grid=(n // _ROWS,),
        in_specs=[
            pl.BlockSpec((_ROWS, din), lambda i: (i, 0)),
            pl.BlockSpec((_ROWS, din), lambda i: (i, 0)),
            pl.BlockSpec((1, din), lambda i: (0, 0)),
            pl.BlockSpec((_ROWS, 1), lambda i: (i, 0)),
            pl.BlockSpec((din, dout), lambda i: (0, 0)),
        ],
        out_specs=pl.BlockSpec((_ROWS, dout), lambda i: (i, 0)),
        out_shape=jax.ShapeDtypeStruct((n, dout), jnp.float32),
    )(agg, hs, b, dinv, w)


def _final_body(a0_ref, a1_ref, hs_ref, b_ref, dinv_ref, eps_ref,
                mu_ref, zeta_ref):
    agg = a0_ref[0] + a1_ref[0]
    mu = dinv_ref[...] * (agg + hs_ref[...]) + b_ref[...]
    mu_ref[...] = mu
    zeta_ref[...] = mu + eps_ref[...] * jnp.exp(mu)


def _final(agg2, hs, b, dinv, eps):
    n, d = hs.shape
    return pl.pallas_call(
        _final_body,
        grid=(n // _ROWS,),
        in_specs=[
            pl.BlockSpec((1, _ROWS, d), lambda i: (0, i, 0)),
            pl.BlockSpec((1, _ROWS, d), lambda i: (1, i, 0)),
            pl.BlockSpec((_ROWS, d), lambda i: (i, 0)),
            pl.BlockSpec((1, d), lambda i: (0, 0)),
            pl.BlockSpec((_ROWS, 1), lambda i: (i, 0)),
            pl.BlockSpec((_ROWS, d), lambda i: (i, 0)),
        ],
        out_specs=[
            pl.BlockSpec((_ROWS, d), lambda i: (i, 0)),
            pl.BlockSpec((_ROWS, d), lambda i: (i, 0)),
        ],
        out_shape=[
            jax.ShapeDtypeStruct((n, d), jnp.float32),
            jax.ShapeDtypeStruct((n, d), jnp.float32),
        ],
    )(agg2, agg2, hs, b, dinv, eps)


def kernel(x, edge_index, W1, b1, Wmu, bmu):
    src = edge_index[0]
    dst = edge_index[1]

    indeg = _deg(dst)
    hs1, dinv = _mm_scale(x, W1, indeg.reshape(_N, 1))
    agg1 = _agg1(hs1, src, dst)
    hs2 = _ep_mm(agg1, hs1, b1[None, :], dinv, Wmu)
    agg2 = _agg2(hs2, src, dst)
    eps = jax.random.normal(jax.random.key(42), hs2.shape, dtype=jnp.float32)
    mu, zeta = _final(agg2, hs2, bmu[None, :], dinv, eps)
    return (mu, mu, zeta)


# speedup vs baseline: 17.0603x; 1.7345x over previous
import functools

import jax
import jax.numpy as jnp
from jax import lax
from jax.experimental import pallas as pl
from jax.experimental.pallas import tpu as pltpu
from jax.experimental.pallas import tpu_sc as plsc

_N = 10000
_E = 320000
_NC = 2
_NS = 16
_K = 80
_ROWS = 1000

_sc_mesh = plsc.VectorSubcoreMesh(core_axis_name="c", subcore_axis_name="s")


def _deg_body(dst_hbm, out_hbm, dstv, ones, zbuf, hist):
    s = lax.axis_index("s")
    c = lax.axis_index("c")
    for i in range(_K // 16):
        ones[pl.ds(i * 16, 16)] = jnp.ones((16,), jnp.float32)
    for i in range(640 // 16):
        zbuf[pl.ds(i * 16, 16)] = jnp.zeros((16,), jnp.float32)
    @pl.when(s < 15)
    def _():
        pltpu.sync_copy(zbuf.at[pl.ds(0, 624)], hist.at[pl.ds(s * 624, 624)])

    @pl.when(s == 15)
    def _():
        pltpu.sync_copy(zbuf, hist.at[pl.ds(9360, 640)])

    plsc.subcore_barrier()
    ep = _E // _NS

    def body(j, carry):
        off = s * ep + j * _K
        pltpu.sync_copy(dst_hbm.at[pl.ds(off, _K)], dstv)
        pltpu.sync_copy(ones, hist.at[dstv], add=True)
        return carry

    lax.fori_loop(0, ep // _K, body, 0)
    plsc.subcore_barrier()
    t = c * _NS + s
    @pl.when(t < 31)
    def _():
        pltpu.sync_copy(hist.at[pl.ds(t * 320, 320)], zbuf.at[pl.ds(0, 320)])
        pltpu.sync_copy(zbuf.at[pl.ds(0, 320)], out_hbm.at[pl.ds(t * 320, 320)])

    @pl.when(t == 31)
    def _():
        pltpu.sync_copy(hist.at[pl.ds(31 * 320, 80)], zbuf.at[pl.ds(0, 80)])
        pltpu.sync_copy(zbuf.at[pl.ds(0, 80)], out_hbm.at[pl.ds(31 * 320, 80)])


_deg = functools.partial(
    pl.kernel,
    _deg_body,
    out_type=jax.ShapeDtypeStruct((_N,), jnp.float32),
    mesh=_sc_mesh,
    scratch_types=[
        pltpu.VMEM((_K,), jnp.int32),
        pltpu.VMEM((_K,), jnp.float32),
        pltpu.VMEM((640,), jnp.float32),
        pltpu.VMEM_SHARED((_N,), jnp.float32),
    ],
)()


def _zero_acc(s, zbuf, acc, fh=128):
    for r in range(8):
        for i in range(fh // 16):
            zbuf[r, pl.ds(i * 16, 16)] = jnp.zeros((16,), jnp.float32)
    nz = jnp.where(s < 15, 78, 80)

    def zbody(j, carry):
        pltpu.sync_copy(zbuf, acc.at[pl.ds(s * 624 + j * 8, 8)])
        return carry

    lax.fori_loop(0, nz, zbody, 0)


_Q = 3


_BN = 25


def _edge_loop(niter, base, table, src_hbm, dst_hbm, srcblk, dstblk,
               svs, dvs, rows, acc, sems):
    def stage_idx(j, p):
        for i in range(_K // 16):
            sl = pl.ds(j * _K + i * 16, 16)
            svs[p][pl.ds(i * 16, 16)] = srcblk[sl]
            dvs[p][pl.ds(i * 16, 16)] = dstblk[sl]

    def gather(p):
        pltpu.async_copy(table.at[svs[p]], rows.at[p], sems[p])

    def wait(p):
        pltpu.make_async_copy(table.at[pl.ds(0, _K)], rows.at[p],
                              sems[p]).wait()

    def scat(p):
        pltpu.sync_copy(rows.at[p], acc.at[dvs[p]], add=True)

    nq, tail = _BN // _Q, _BN % _Q

    def block(b, carry):
        off = base + b * _BN * _K
        pltpu.sync_copy(src_hbm.at[pl.ds(off, _BN * _K)], srcblk)
        pltpu.sync_copy(dst_hbm.at[pl.ds(off, _BN * _K)], dstblk)

        def body(jj, carry2):
            j0 = jj * _Q
            for p in range(_Q):
                stage_idx(j0 + p, p)
                gather(p)
            for p in range(_Q):
                wait(p)
                scat(p)
            return carry2

        lax.fori_loop(0, nq, body, 0)
        for p in range(tail):
            stage_idx(nq * _Q + p, p)
            gather(p)
        for p in range(tail):
            wait(p)
            scat(p)
        return carry

    lax.fori_loop(0, niter // _BN, block, 0)


def _drain_acc(s, stage, acc, out2d):
    def obody(j, carry):
        r = s * 624 + j * 80
        pltpu.sync_copy(acc.at[pl.ds(r, 80)], stage)
        pltpu.sync_copy(stage, out2d.at[pl.ds(r, 80)])
        return carry

    lax.fori_loop(0, 7, obody, 0)
    @pl.when(s < 15)
    def _():
        r = s * 624 + 560
        pltpu.sync_copy(acc.at[pl.ds(r, 64)], stage.at[pl.ds(0, 64)])
        pltpu.sync_copy(stage.at[pl.ds(0, 64)], out2d.at[pl.ds(r, 64)])

    @pl.when(s == 15)
    def _():
        pltpu.sync_copy(acc.at[pl.ds(9920, 80)], stage)
        pltpu.sync_copy(stage, out2d.at[pl.ds(9920, 80)])


def _xform(srcall, ne, mul, add):
    def xb(i, carry):
        sl = pl.ds(i * 16, 16)
        srcall[sl] = srcall[sl] * mul + add
        return carry

    lax.fori_loop(0, ne // 16, xb, 0)


def _agg1_body(table_hbm, src_hbm, dst_hbm, out_hbm, srcall, dstall, *rest):
    svs, dvs = rest[0:_Q], rest[_Q:2 * _Q]
    rows, zbuf, acc = rest[2 * _Q:2 * _Q + 3]
    sems = rest[2 * _Q + 3:]
    s = lax.axis_index("s")
    c = lax.axis_index("c")
    ep = _E // _NS
    _zero_acc(s, zbuf, acc)
    plsc.subcore_barrier()
    col = pl.multiple_of(c * 128, 128)
    tcol = table_hbm.at[:, pl.ds(col, 128)]
    _edge_loop(ep // _K, s * ep, tcol, src_hbm, dst_hbm, srcall, dstall,
               svs, dvs, rows, acc, sems)
    plsc.subcore_barrier()
    _drain_acc(s, rows.at[0], acc, out_hbm.at[:, pl.ds(col, 128)])


def _agg_scratch(ne):
    del ne
    return (
        [pltpu.VMEM((_BN * _K,), jnp.int32), pltpu.VMEM((_BN * _K,), jnp.int32)]
        + [pltpu.VMEM((_K,), jnp.int32) for _ in range(2 * _Q)]
        + [
            pltpu.VMEM((_Q, _K, 128), jnp.float32),
            pltpu.VMEM((8, 128), jnp.float32),
            pltpu.VMEM_SHARED((_N, 128), jnp.float32),
        ]
        + [pltpu.SemaphoreType.DMA for _ in range(_Q)]
    )


_agg1 = functools.partial(
    pl.kernel,
    _agg1_body,
    out_type=jax.ShapeDtypeStruct((_N, 256), jnp.float32),
    mesh=_sc_mesh,
    scratch_types=_agg_scratch(_E // _NS),
)()


def _agg2_body(table_hbm, src_hbm, dst_hbm, out_hbm, srcall, dstall, *rest):
    svs, dvs = rest[0:_Q], rest[_Q:2 * _Q]
    rows, zbuf, acc = rest[2 * _Q:2 * _Q + 3]
    sems = rest[2 * _Q + 3:]
    s = lax.axis_index("s")
    c = lax.axis_index("c")
    ep = _E // (_NC * _NS)
    base = (c * _NS + s) * ep
    _zero_acc(s, zbuf, acc)
    plsc.subcore_barrier()
    _edge_loop(ep // _K, base, table_hbm, src_hbm, dst_hbm, srcall, dstall,
               svs, dvs, rows, acc, sems)
    plsc.subcore_barrier()
    _drain_acc(s, rows.at[0], acc, out_hbm.at[c])


_agg2 = functools.partial(
    pl.kernel,
    _agg2_body,
    out_type=jax.ShapeDtypeStruct((_NC, _N, 128), jnp.float32),
    mesh=_sc_mesh,
    scratch_types=_agg_scratch(_E // (_NC * _NS)),
)()


def _mm_scale_body(x_ref, w_ref, indeg_ref, hs_ref, dinv_ref):
    dinv = lax.rsqrt(indeg_ref[...] + 1.0)
    h = jnp.dot(x_ref[...], w_ref[...], preferred_element_type=jnp.float32)
    hs_ref[...] = h * dinv
    dinv_ref[...] = dinv


def _mm_scale(x, w, indeg):
    n, _ = x.shape
    dout = w.shape[1]
    return pl.pallas_call(
        _mm_scale_body,
        grid=(n // _ROWS,),
        in_specs=[
            pl.BlockSpec((_ROWS, x.shape[1]), lambda i: (i, 0)),
            pl.BlockSpec((w.shape[0], dout), lambda i: (0, 0)),
            pl.BlockSpec((_ROWS, 1), lambda i: (i, 0)),
        ],
        out_specs=[
            pl.BlockSpec((_ROWS, dout), lambda i: (i, 0)),
            pl.BlockSpec((_ROWS, 1), lambda i: (i, 0)),
        ],
        out_shape=[
            jax.ShapeDtypeStruct((n, dout), jnp.float32),
            jax.ShapeDtypeStruct((n, 1), jnp.float32),
        ],
    )(x, w, indeg)


def _ep_mm_body(agg_ref, hs_ref, b_ref, dinv_ref, w_ref, out_ref):
    dinv = dinv_ref[...]
    h = jnp.maximum(dinv * (agg_ref[...] + hs_ref[...]) + b_ref[...], 0.0)
    out_ref[...] = jnp.dot(h, w_ref[...], preferred_element_type=jnp.float32) * dinv


def _ep_mm(agg, hs, b, dinv, w):
    n, din = hs.shape
    dout = w.shape[1]
    return pl.pallas_call(
        _ep_mm_body,
        grid=(n // _ROWS,),
        in_specs=[
            pl.BlockSpec((_ROWS, din), lambda i: (i, 0)),
            pl.BlockSpec((_ROWS, din), lambda i: (i, 0)),
            pl.BlockSpec((1, din), lambda i: (0, 0)),
            pl.BlockSpec((_ROWS, 1), lambda i: (i, 0)),
            pl.BlockSpec((din, dout), lambda i: (0, 0)),
        ],
        out_specs=pl.BlockSpec((_ROWS, dout), lambda i: (i, 0)),
        out_shape=jax.ShapeDtypeStruct((n, dout), jnp.float32),
    )(agg, hs, b, dinv, w)


def _final_body(a0_ref, a1_ref, hs_ref, b_ref, dinv_ref, eps_ref,
                mu_ref, zeta_ref):
    agg = a0_ref[0] + a1_ref[0]
    mu = dinv_ref[...] * (agg + hs_ref[...]) + b_ref[...]
    mu_ref[...] = mu
    zeta_ref[...] = mu + eps_ref[...] * jnp.exp(mu)


def _final(agg2, hs, b, dinv, eps):
    n, d = hs.shape
    return pl.pallas_call(
        _final_body,
        grid=(n // _ROWS,),
        in_specs=[
            pl.BlockSpec((1, _ROWS, d), lambda i: (0, i, 0)),
            pl.BlockSpec((1, _ROWS, d), lambda i: (1, i, 0)),
            pl.BlockSpec((_ROWS, d), lambda i: (i, 0)),
            pl.BlockSpec((1, d), lambda i: (0, 0)),
            pl.BlockSpec((_ROWS, 1), lambda i: (i, 0)),
            pl.BlockSpec((_ROWS, d), lambda i: (i, 0)),
        ],
        out_specs=[
            pl.BlockSpec((_ROWS, d), lambda i: (i, 0)),
            pl.BlockSpec((_ROWS, d), lambda i: (i, 0)),
        ],
        out_shape=[
            jax.ShapeDtypeStruct((n, d), jnp.float32),
            jax.ShapeDtypeStruct((n, d), jnp.float32),
        ],
    )(agg2, agg2, hs, b, dinv, eps)


def kernel(x, edge_index, W1, b1, Wmu, bmu):
    src = edge_index[0]
    dst = edge_index[1]

    indeg = _deg(dst)
    hs1, dinv = _mm_scale(x, W1, indeg.reshape(_N, 1))
    agg1 = _agg1(hs1, src, dst)
    hs2 = _ep_mm(agg1, hs1, b1[None, :], dinv, Wmu)
    agg2 = _agg2(hs2, src, dst)
    eps = jax.random.normal(jax.random.key(42), hs2.shape, dtype=jnp.float32)
    mu, zeta = _final(agg2, hs2, bmu[None, :], dinv, eps)
    return (mu, mu, zeta)


# speedup vs baseline: 20.3515x; 1.1929x over previous
import functools

import jax
import jax.numpy as jnp
from jax import lax
from jax.experimental import pallas as pl
from jax.experimental.pallas import tpu as pltpu
from jax.experimental.pallas import tpu_sc as plsc

_N = 10000
_E = 320000
_NC = 2
_NS = 16
_K = 80
_BN = 25
_ROWS = 1000

_sc_mesh = plsc.VectorSubcoreMesh(core_axis_name="c", subcore_axis_name="s")


def _deg_body(dst_hbm, out_hbm, dstblk, dv0, dv1, ones, zbuf, hist, sm0, sm1):
    s = lax.axis_index("s")
    c = lax.axis_index("c")
    for i in range(_K // 16):
        ones[pl.ds(i * 16, 16)] = jnp.ones((16,), jnp.float32)
    for i in range(640 // 16):
        zbuf[pl.ds(i * 16, 16)] = jnp.zeros((16,), jnp.float32)
    @pl.when(s < 15)
    def _():
        pltpu.sync_copy(zbuf.at[pl.ds(0, 624)], hist.at[pl.ds(s * 624, 624)])

    @pl.when(s == 15)
    def _():
        pltpu.sync_copy(zbuf, hist.at[pl.ds(9360, 640)])

    plsc.subcore_barrier()
    ep = _E // _NS
    dvs, sems = (dv0, dv1), (sm0, sm1)

    def stage(j, p):
        for i in range(_K // 16):
            dvs[p][pl.ds(i * 16, 16)] = dstblk[pl.ds(j * _K + i * 16, 16)]

    def block(b, carry):
        pltpu.sync_copy(dst_hbm.at[pl.ds(s * ep + b * _BN * _K, _BN * _K)],
                        dstblk)

        def body(jj, carry2):
            for p in range(2):
                stage(jj * 2 + p, p)
                pltpu.async_copy(ones, hist.at[dvs[p]], sems[p], add=True)
            for p in range(2):
                pltpu.make_async_copy(ones, hist.at[dvs[p]], sems[p]).wait()
            return carry2

        lax.fori_loop(0, _BN // 2, body, 0)
        stage(_BN - 1, 0)
        pltpu.sync_copy(ones, hist.at[dvs[0]], add=True)
        return carry

    lax.fori_loop(0, ep // (_BN * _K), block, 0)
    plsc.subcore_barrier()
    t = c * _NS + s
    @pl.when(t < 31)
    def _():
        pltpu.sync_copy(hist.at[pl.ds(t * 320, 320)], zbuf.at[pl.ds(0, 320)])
        pltpu.sync_copy(zbuf.at[pl.ds(0, 320)], out_hbm.at[pl.ds(t * 320, 320)])

    @pl.when(t == 31)
    def _():
        pltpu.sync_copy(hist.at[pl.ds(31 * 320, 80)], zbuf.at[pl.ds(0, 80)])
        pltpu.sync_copy(zbuf.at[pl.ds(0, 80)], out_hbm.at[pl.ds(31 * 320, 80)])


_deg = functools.partial(
    pl.kernel,
    _deg_body,
    out_type=jax.ShapeDtypeStruct((_N,), jnp.float32),
    mesh=_sc_mesh,
    scratch_types=[
        pltpu.VMEM((_BN * _K,), jnp.int32),
        pltpu.VMEM((_K,), jnp.int32),
        pltpu.VMEM((_K,), jnp.int32),
        pltpu.VMEM((_K,), jnp.float32),
        pltpu.VMEM((640,), jnp.float32),
        pltpu.VMEM_SHARED((_N,), jnp.float32),
        pltpu.SemaphoreType.DMA,
        pltpu.SemaphoreType.DMA,
    ],
)()


def _zero_acc(s, zbuf, acc, fh=128):
    for r in range(8):
        for i in range(fh // 16):
            zbuf[r, pl.ds(i * 16, 16)] = jnp.zeros((16,), jnp.float32)
    nz = jnp.where(s < 15, 78, 80)

    def zbody(j, carry):
        pltpu.sync_copy(zbuf, acc.at[pl.ds(s * 624 + j * 8, 8)])
        return carry

    lax.fori_loop(0, nz, zbody, 0)


_Q = 3


def _edge_loop(niter, base, table, src_hbm, dst_hbm, srcblk, dstblk,
               svs, dvs, rows, acc, sems):
    def stage_idx(j, p):
        for i in range(_K // 16):
            sl = pl.ds(j * _K + i * 16, 16)
            svs[p][pl.ds(i * 16, 16)] = srcblk[sl]
            dvs[p][pl.ds(i * 16, 16)] = dstblk[sl]

    def gather(p):
        pltpu.async_copy(table.at[svs[p]], rows.at[p], sems[p])

    def wait(p):
        pltpu.make_async_copy(table.at[pl.ds(0, _K)], rows.at[p],
                              sems[p]).wait()

    def scat(p):
        pltpu.sync_copy(rows.at[p], acc.at[dvs[p]], add=True)

    nq, tail = _BN // _Q, _BN % _Q

    def block(b, carry):
        off = base + b * _BN * _K
        pltpu.sync_copy(src_hbm.at[pl.ds(off, _BN * _K)], srcblk)
        pltpu.sync_copy(dst_hbm.at[pl.ds(off, _BN * _K)], dstblk)

        def body(jj, carry2):
            j0 = jj * _Q
            for p in range(_Q):
                stage_idx(j0 + p, p)
                gather(p)
            for p in range(_Q):
                wait(p)
                scat(p)
            return carry2

        lax.fori_loop(0, nq, body, 0)
        for p in range(tail):
            stage_idx(nq * _Q + p, p)
            gather(p)
        for p in range(tail):
            wait(p)
            scat(p)
        return carry

    lax.fori_loop(0, niter // _BN, block, 0)


def _drain_acc(s, stage, acc, out2d):
    def obody(j, carry):
        r = s * 624 + j * 80
        pltpu.sync_copy(acc.at[pl.ds(r, 80)], stage)
        pltpu.sync_copy(stage, out2d.at[pl.ds(r, 80)])
        return carry

    lax.fori_loop(0, 7, obody, 0)
    @pl.when(s < 15)
    def _():
        r = s * 624 + 560
        pltpu.sync_copy(acc.at[pl.ds(r, 64)], stage.at[pl.ds(0, 64)])
        pltpu.sync_copy(stage.at[pl.ds(0, 64)], out2d.at[pl.ds(r, 64)])

    @pl.when(s == 15)
    def _():
        pltpu.sync_copy(acc.at[pl.ds(9920, 80)], stage)
        pltpu.sync_copy(stage, out2d.at[pl.ds(9920, 80)])


def _xform(srcall, ne, mul, add):
    def xb(i, carry):
        sl = pl.ds(i * 16, 16)
        srcall[sl] = srcall[sl] * mul + add
        return carry

    lax.fori_loop(0, ne // 16, xb, 0)


def _agg1_body(table_hbm, src_hbm, dst_hbm, out_hbm, srcall, dstall, *rest):
    svs, dvs = rest[0:_Q], rest[_Q:2 * _Q]
    rows, zbuf, acc = rest[2 * _Q:2 * _Q + 3]
    sems = rest[2 * _Q + 3:]
    s = lax.axis_index("s")
    c = lax.axis_index("c")
    ep = _E // _NS
    _zero_acc(s, zbuf, acc)
    plsc.subcore_barrier()
    col = pl.multiple_of(c * 128, 128)
    tcol = table_hbm.at[:, pl.ds(col, 128)]
    _edge_loop(ep // _K, s * ep, tcol, src_hbm, dst_hbm, srcall, dstall,
               svs, dvs, rows, acc, sems)
    plsc.subcore_barrier()
    _drain_acc(s, rows.at[0], acc, out_hbm.at[:, pl.ds(col, 128)])


def _agg_scratch(ne):
    del ne
    return (
        [pltpu.VMEM((_BN * _K,), jnp.int32), pltpu.VMEM((_BN * _K,), jnp.int32)]
        + [pltpu.VMEM((_K,), jnp.int32) for _ in range(2 * _Q)]
        + [
            pltpu.VMEM((_Q, _K, 128), jnp.float32),
            pltpu.VMEM((8, 128), jnp.float32),
            pltpu.VMEM_SHARED((_N, 128), jnp.float32),
        ]
        + [pltpu.SemaphoreType.DMA for _ in range(_Q)]
    )


_agg1 = functools.partial(
    pl.kernel,
    _agg1_body,
    out_type=jax.ShapeDtypeStruct((_N, 256), jnp.float32),
    mesh=_sc_mesh,
    scratch_types=_agg_scratch(_E // _NS),
)()


def _agg2_body(table_hbm, src_hbm, dst_hbm, out_hbm, srcall, dstall, *rest):
    svs, dvs = rest[0:_Q], rest[_Q:2 * _Q]
    rows, zbuf, acc = rest[2 * _Q:2 * _Q + 3]
    sems = rest[2 * _Q + 3:]
    s = lax.axis_index("s")
    c = lax.axis_index("c")
    ep = _E // (_NC * _NS)
    base = (c * _NS + s) * ep
    _zero_acc(s, zbuf, acc)
    plsc.subcore_barrier()
    _edge_loop(ep // _K, base, table_hbm, src_hbm, dst_hbm, srcall, dstall,
               svs, dvs, rows, acc, sems)
    plsc.subcore_barrier()
    _drain_acc(s, rows.at[0], acc, out_hbm.at[c])


_agg2 = functools.partial(
    pl.kernel,
    _agg2_body,
    out_type=jax.ShapeDtypeStruct((_NC, _N, 128), jnp.float32),
    mesh=_sc_mesh,
    scratch_types=_agg_scratch(_E // (_NC * _NS)),
)()


def _mm_scale_body(x_ref, w_ref, indeg_ref, hs_ref, dinv_ref):
    dinv = lax.rsqrt(indeg_ref[...] + 1.0)
    h = jnp.dot(x_ref[...], w_ref[...], preferred_element_type=jnp.float32)
    hs_ref[...] = h * dinv
    dinv_ref[...] = dinv


def _mm_scale(x, w, indeg):
    n, _ = x.shape
    dout = w.shape[1]
    return pl.pallas_call(
        _mm_scale_body,
        grid=(n // _ROWS,),
        in_specs=[
            pl.BlockSpec((_ROWS, x.shape[1]), lambda i: (i, 0)),
            pl.BlockSpec((w.shape[0], dout), lambda i: (0, 0)),
            pl.BlockSpec((_ROWS, 1), lambda i: (i, 0)),
        ],
        out_specs=[
            pl.BlockSpec((_ROWS, dout), lambda i: (i, 0)),
            pl.BlockSpec((_ROWS, 1), lambda i: (i, 0)),
        ],
        out_shape=[
            jax.ShapeDtypeStruct((n, dout), jnp.float32),
            jax.ShapeDtypeStruct((n, 1), jnp.float32),
        ],
    )(x, w, indeg)


def _ep_mm_body(agg_ref, hs_ref, b_ref, dinv_ref, w_ref, out_ref):
    dinv = dinv_ref[...]
    h = jnp.maximum(dinv * (agg_ref[...] + hs_ref[...]) + b_ref[...], 0.0)
    out_ref[...] = jnp.dot(h, w_ref[...], preferred_element_type=jnp.float32) * dinv


def _ep_mm(agg, hs, b, dinv, w):
    n, din = hs.shape
    dout = w.shape[1]
    return pl.pallas_call(
        _ep_mm_body,
        grid=(n // _ROWS,),
        in_specs=[
            pl.BlockSpec((_ROWS, din), lambda i: (i, 0)),
            pl.BlockSpec((_ROWS, din), lambda i: (i, 0)),
            pl.BlockSpec((1, din), lambda i: (0, 0)),
            pl.BlockSpec((_ROWS, 1), lambda i: (i, 0)),
            pl.BlockSpec((din, dout), lambda i: (0, 0)),
        ],
        out_specs=pl.BlockSpec((_ROWS, dout), lambda i: (i, 0)),
        out_shape=jax.ShapeDtypeStruct((n, dout), jnp.float32),
    )(agg, hs, b, dinv, w)


def _final_body(a0_ref, a1_ref, hs_ref, b_ref, dinv_ref, eps_ref,
                mu_ref, zeta_ref):
    agg = a0_ref[0] + a1_ref[0]
    mu = dinv_ref[...] * (agg + hs_ref[...]) + b_ref[...]
    mu_ref[...] = mu
    zeta_ref[...] = mu + eps_ref[...] * jnp.exp(mu)


def _final(agg2, hs, b, dinv, eps):
    n, d = hs.shape
    return pl.pallas_call(
        _final_body,
        grid=(n // _ROWS,),
        in_specs=[
            pl.BlockSpec((1, _ROWS, d), lambda i: (0, i, 0)),
            pl.BlockSpec((1, _ROWS, d), lambda i: (1, i, 0)),
            pl.BlockSpec((_ROWS, d), lambda i: (i, 0)),
            pl.BlockSpec((1, d), lambda i: (0, 0)),
            pl.BlockSpec((_ROWS, 1), lambda i: (i, 0)),
            pl.BlockSpec((_ROWS, d), lambda i: (i, 0)),
        ],
        out_specs=[
            pl.BlockSpec((_ROWS, d), lambda i: (i, 0)),
            pl.BlockSpec((_ROWS, d), lambda i: (i, 0)),
        ],
        out_shape=[
            jax.ShapeDtypeStruct((n, d), jnp.float32),
            jax.ShapeDtypeStruct((n, d), jnp.float32),
        ],
    )(agg2, agg2, hs, b, dinv, eps)


def kernel(x, edge_index, W1, b1, Wmu, bmu):
    src = edge_index[0]
    dst = edge_index[1]

    indeg = _deg(dst)
    hs1, dinv = _mm_scale(x, W1, indeg.reshape(_N, 1))
    agg1 = _agg1(hs1, src, dst)
    hs2 = _ep_mm(agg1, hs1, b1[None, :], dinv, Wmu)
    agg2 = _agg2(hs2, src, dst)
    eps = jax.random.normal(jax.random.key(42), hs2.shape, dtype=jnp.float32)
    mu, zeta = _final(agg2, hs2, bmu[None, :], dinv, eps)
    return (mu, mu, zeta)


# speedup vs baseline: 20.8734x; 1.0256x over previous
import functools

import jax
import jax.numpy as jnp
from jax import lax
from jax.experimental import pallas as pl
from jax.experimental.pallas import tpu as pltpu
from jax.experimental.pallas import tpu_sc as plsc

_N = 10000
_E = 320000
_NC = 2
_NS = 16
_K = 80
_BN = 25
_ROWS = 1000

_sc_mesh = plsc.VectorSubcoreMesh(core_axis_name="c", subcore_axis_name="s")


def _deg_body(dst_hbm, out_hbm, dstblk, dv0, dv1, ones, zbuf, hist, sm0, sm1):
    s = lax.axis_index("s")
    c = lax.axis_index("c")
    for i in range(_K // 16):
        ones[pl.ds(i * 16, 16)] = jnp.ones((16,), jnp.float32)
    for i in range(640 // 16):
        zbuf[pl.ds(i * 16, 16)] = jnp.zeros((16,), jnp.float32)
    @pl.when(s < 15)
    def _():
        pltpu.sync_copy(zbuf.at[pl.ds(0, 624)], hist.at[pl.ds(s * 624, 624)])

    @pl.when(s == 15)
    def _():
        pltpu.sync_copy(zbuf, hist.at[pl.ds(9360, 640)])

    plsc.subcore_barrier()
    ep = _E // _NS
    dvs, sems = (dv0, dv1), (sm0, sm1)

    def stage(j, p):
        for i in range(_K // 16):
            dvs[p][pl.ds(i * 16, 16)] = dstblk[pl.ds(j * _K + i * 16, 16)]

    def block(b, carry):
        pltpu.sync_copy(dst_hbm.at[pl.ds(s * ep + b * _BN * _K, _BN * _K)],
                        dstblk)

        def body(jj, carry2):
            for p in range(2):
                stage(jj * 2 + p, p)
                pltpu.async_copy(ones, hist.at[dvs[p]], sems[p], add=True)
            for p in range(2):
                pltpu.make_async_copy(ones, hist.at[dvs[p]], sems[p]).wait()
            return carry2

        lax.fori_loop(0, _BN // 2, body, 0)
        stage(_BN - 1, 0)
        pltpu.sync_copy(ones, hist.at[dvs[0]], add=True)
        return carry

    lax.fori_loop(0, ep // (_BN * _K), block, 0)
    plsc.subcore_barrier()
    t = c * _NS + s
    @pl.when(t < 31)
    def _():
        pltpu.sync_copy(hist.at[pl.ds(t * 320, 320)], zbuf.at[pl.ds(0, 320)])
        pltpu.sync_copy(zbuf.at[pl.ds(0, 320)], out_hbm.at[pl.ds(t * 320, 320)])

    @pl.when(t == 31)
    def _():
        pltpu.sync_copy(hist.at[pl.ds(31 * 320, 80)], zbuf.at[pl.ds(0, 80)])
        pltpu.sync_copy(zbuf.at[pl.ds(0, 80)], out_hbm.at[pl.ds(31 * 320, 80)])


_deg = functools.partial(
    pl.kernel,
    _deg_body,
    out_type=jax.ShapeDtypeStruct((_N,), jnp.float32),
    mesh=_sc_mesh,
    scratch_types=[
        pltpu.VMEM((_BN * _K,), jnp.int32),
        pltpu.VMEM((_K,), jnp.int32),
        pltpu.VMEM((_K,), jnp.int32),
        pltpu.VMEM((_K,), jnp.float32),
        pltpu.VMEM((640,), jnp.float32),
        pltpu.VMEM_SHARED((_N,), jnp.float32),
        pltpu.SemaphoreType.DMA,
        pltpu.SemaphoreType.DMA,
    ],
)()


def _zero_acc(s, zbuf, acc, fh=128):
    for r in range(8):
        for i in range(fh // 16):
            zbuf[r, pl.ds(i * 16, 16)] = jnp.zeros((16,), jnp.float32)
    nz = jnp.where(s < 15, 78, 80)

    def zbody(j, carry):
        pltpu.sync_copy(zbuf, acc.at[pl.ds(s * 624 + j * 8, 8)])
        return carry

    lax.fori_loop(0, nz, zbody, 0)


_Q = 4


def _edge_loop(niter, base, table, src_hbm, dst_hbm, srcblk, dstblk,
               svs, dvs, rows, acc, sems):
    def stage_idx(j, p):
        for i in range(_K // 16):
            sl = pl.ds(j * _K + i * 16, 16)
            svs[p][pl.ds(i * 16, 16)] = srcblk[sl]
            dvs[p][pl.ds(i * 16, 16)] = dstblk[sl]

    def gather(p):
        pltpu.async_copy(table.at[svs[p]], rows.at[p], sems[p])

    def wait(p):
        pltpu.make_async_copy(table.at[pl.ds(0, _K)], rows.at[p],
                              sems[p]).wait()

    def scat(p):
        pltpu.sync_copy(rows.at[p], acc.at[dvs[p]], add=True)

    nq, tail = _BN // _Q, _BN % _Q

    def block(b, carry):
        off = base + b * _BN * _K
        pltpu.sync_copy(src_hbm.at[pl.ds(off, _BN * _K)], srcblk)
        pltpu.sync_copy(dst_hbm.at[pl.ds(off, _BN * _K)], dstblk)

        def body(jj, carry2):
            j0 = jj * _Q
            for p in range(_Q):
                stage_idx(j0 + p, p)
                gather(p)
            for p in range(_Q):
                wait(p)
                scat(p)
            return carry2

        lax.fori_loop(0, nq, body, 0)
        for p in range(tail):
            stage_idx(nq * _Q + p, p)
            gather(p)
        for p in range(tail):
            wait(p)
            scat(p)
        return carry

    lax.fori_loop(0, niter // _BN, block, 0)


def _drain_acc(s, stage, acc, out2d):
    def obody(j, carry):
        r = s * 624 + j * 80
        pltpu.sync_copy(acc.at[pl.ds(r, 80)], stage)
        pltpu.sync_copy(stage, out2d.at[pl.ds(r, 80)])
        return carry

    lax.fori_loop(0, 7, obody, 0)
    @pl.when(s < 15)
    def _():
        r = s * 624 + 560
        pltpu.sync_copy(acc.at[pl.ds(r, 64)], stage.at[pl.ds(0, 64)])
        pltpu.sync_copy(stage.at[pl.ds(0, 64)], out2d.at[pl.ds(r, 64)])

    @pl.when(s == 15)
    def _():
        pltpu.sync_copy(acc.at[pl.ds(9920, 80)], stage)
        pltpu.sync_copy(stage, out2d.at[pl.ds(9920, 80)])


def _xform(srcall, ne, mul, add):
    def xb(i, carry):
        sl = pl.ds(i * 16, 16)
        srcall[sl] = srcall[sl] * mul + add
        return carry

    lax.fori_loop(0, ne // 16, xb, 0)


def _agg1_body(table_hbm, src_hbm, dst_hbm, out_hbm, srcall, dstall, *rest):
    svs, dvs = rest[0:_Q], rest[_Q:2 * _Q]
    rows, zbuf, acc = rest[2 * _Q:2 * _Q + 3]
    sems = rest[2 * _Q + 3:]
    s = lax.axis_index("s")
    c = lax.axis_index("c")
    ep = _E // _NS
    _zero_acc(s, zbuf, acc)
    plsc.subcore_barrier()
    col = pl.multiple_of(c * 128, 128)
    tcol = table_hbm.at[:, pl.ds(col, 128)]
    _edge_loop(ep // _K, s * ep, tcol, src_hbm, dst_hbm, srcall, dstall,
               svs, dvs, rows, acc, sems)
    plsc.subcore_barrier()
    _drain_acc(s, rows.at[0], acc, out_hbm.at[:, pl.ds(col, 128)])


def _agg_scratch(ne):
    del ne
    return (
        [pltpu.VMEM((_BN * _K,), jnp.int32), pltpu.VMEM((_BN * _K,), jnp.int32)]
        + [pltpu.VMEM((_K,), jnp.int32) for _ in range(2 * _Q)]
        + [
            pltpu.VMEM((_Q, _K, 128), jnp.float32),
            pltpu.VMEM((8, 128), jnp.float32),
            pltpu.VMEM_SHARED((_N, 128), jnp.float32),
        ]
        + [pltpu.SemaphoreType.DMA for _ in range(_Q)]
    )


_agg1 = functools.partial(
    pl.kernel,
    _agg1_body,
    out_type=jax.ShapeDtypeStruct((_N, 256), jnp.float32),
    mesh=_sc_mesh,
    scratch_types=_agg_scratch(_E // _NS),
)()


def _agg2_body(table_hbm, src_hbm, dst_hbm, out_hbm, srcall, dstall, *rest):
    svs, dvs = rest[0:_Q], rest[_Q:2 * _Q]
    rows, zbuf, acc = rest[2 * _Q:2 * _Q + 3]
    sems = rest[2 * _Q + 3:]
    s = lax.axis_index("s")
    c = lax.axis_index("c")
    ep = _E // (_NC * _NS)
    base = (c * _NS + s) * ep
    _zero_acc(s, zbuf, acc)
    plsc.subcore_barrier()
    _edge_loop(ep // _K, base, table_hbm, src_hbm, dst_hbm, srcall, dstall,
               svs, dvs, rows, acc, sems)
    plsc.subcore_barrier()
    _drain_acc(s, rows.at[0], acc, out_hbm.at[c])


_agg2 = functools.partial(
    pl.kernel,
    _agg2_body,
    out_type=jax.ShapeDtypeStruct((_NC, _N, 128), jnp.float32),
    mesh=_sc_mesh,
    scratch_types=_agg_scratch(_E // (_NC * _NS)),
)()


def _mm_body(x_ref, w_ref, h_ref):
    h_ref[...] = jnp.dot(x_ref[...], w_ref[...],
                         preferred_element_type=jnp.float32)


def _mm(x, w):
    n, _ = x.shape
    dout = w.shape[1]
    return pl.pallas_call(
        _mm_body,
        grid=(n // _ROWS,),
        in_specs=[
            pl.BlockSpec((_ROWS, x.shape[1]), lambda i: (i, 0)),
            pl.BlockSpec((w.shape[0], dout), lambda i: (0, 0)),
        ],
        out_specs=pl.BlockSpec((_ROWS, dout), lambda i: (i, 0)),
        out_shape=jax.ShapeDtypeStruct((n, dout), jnp.float32),
    )(x, w)


def _scale_body(h_ref, indeg_ref, hs_ref, dinv_ref):
    dinv = lax.rsqrt(indeg_ref[...] + 1.0)
    hs_ref[...] = h_ref[...] * dinv
    dinv_ref[...] = dinv


def _scale(h, indeg):
    n, dout = h.shape
    return pl.pallas_call(
        _scale_body,
        grid=(n // _ROWS,),
        in_specs=[
            pl.BlockSpec((_ROWS, dout), lambda i: (i, 0)),
            pl.BlockSpec((_ROWS, 1), lambda i: (i, 0)),
        ],
        out_specs=[
            pl.BlockSpec((_ROWS, dout), lambda i: (i, 0)),
            pl.BlockSpec((_ROWS, 1), lambda i: (i, 0)),
        ],
        out_shape=[
            jax.ShapeDtypeStruct((n, dout), jnp.float32),
            jax.ShapeDtypeStruct((n, 1), jnp.float32),
        ],
    )(h, indeg)


def _ep_mm_body(agg_ref, hs_ref, b_ref, dinv_ref, w_ref, out_ref):
    dinv = dinv_ref[...]
    h = jnp.maximum(dinv * (agg_ref[...] + hs_ref[...]) + b_ref[...], 0.0)
    out_ref[...] = jnp.dot(h, w_ref[...], preferred_element_type=jnp.float32) * dinv


def _ep_mm(agg, hs, b, dinv, w):
    n, din = hs.shape
    dout = w.shape[1]
    return pl.pallas_call(
        _ep_mm_body,
        grid=(n // _ROWS,),
        in_specs=[
            pl.BlockSpec((_ROWS, din), lambda i: (i, 0)),
            pl.BlockSpec((_ROWS, din), lambda i: (i, 0)),
            pl.BlockSpec((1, din), lambda i: (0, 0)),
            pl.BlockSpec((_ROWS, 1), lambda i: (i, 0)),
            pl.BlockSpec((din, dout), lambda i: (0, 0)),
        ],
        out_specs=pl.BlockSpec((_ROWS, dout), lambda i: (i, 0)),
        out_shape=jax.ShapeDtypeStruct((n, dout), jnp.float32),
    )(agg, hs, b, dinv, w)


def _final_body(a0_ref, a1_ref, hs_ref, b_ref, dinv_ref, eps_ref,
                mu_ref, zeta_ref):
    agg = a0_ref[0] + a1_ref[0]
    mu = dinv_ref[...] * (agg + hs_ref[...]) + b_ref[...]
    mu_ref[...] = mu
    zeta_ref[...] = mu + eps_ref[...] * jnp.exp(mu)


def _final(agg2, hs, b, dinv, eps):
    n, d = hs.shape
    return pl.pallas_call(
        _final_body,
        grid=(n // _ROWS,),
        in_specs=[
            pl.BlockSpec((1, _ROWS, d), lambda i: (0, i, 0)),
            pl.BlockSpec((1, _ROWS, d), lambda i: (1, i, 0)),
            pl.BlockSpec((_ROWS, d), lambda i: (i, 0)),
            pl.BlockSpec((1, d), lambda i: (0, 0)),
            pl.BlockSpec((_ROWS, 1), lambda i: (i, 0)),
            pl.BlockSpec((_ROWS, d), lambda i: (i, 0)),
        ],
        out_specs=[
            pl.BlockSpec((_ROWS, d), lambda i: (i, 0)),
            pl.BlockSpec((_ROWS, d), lambda i: (i, 0)),
        ],
        out_shape=[
            jax.ShapeDtypeStruct((n, d), jnp.float32),
            jax.ShapeDtypeStruct((n, d), jnp.float32),
        ],
    )(agg2, agg2, hs, b, dinv, eps)


def kernel(x, edge_index, W1, b1, Wmu, bmu):
    src = edge_index[0]
    dst = edge_index[1]

    h1 = _mm(x, W1)
    indeg = _deg(dst)
    hs1, dinv = _scale(h1, indeg.reshape(_N, 1))
    agg1 = _agg1(hs1, src, dst)
    hs2 = _ep_mm(agg1, hs1, b1[None, :], dinv, Wmu)
    agg2 = _agg2(hs2, src, dst)
    eps = jax.random.normal(jax.random.key(42), hs2.shape, dtype=jnp.float32)
    mu, zeta = _final(agg2, hs2, bmu[None, :], dinv, eps)
    return (mu, mu, zeta)
